# baseline (device time: 71894 ns/iter reference)
import jax
import jax.numpy as jnp
from jax import lax
from jax.experimental import pallas as pl
from jax.experimental.pallas import tpu as pltpu

N_DEV = 8
SQ_BLK = 256
HALF = SQ_BLK // 2
D_MODEL = 1024
H_PER = 8
DH = 128
WIN = 512
K_ROWS = 2176
SCALE = 0.08838834764831843
NEG = -1e9
BF = jnp.bfloat16


def _body(x_ref, wq_ref, k_hbm, v_hbm, wo_ref, out_ref,
          kf32, vf32, kb, vb, xb, wqb, wob, xl, xr, aacc, bacc,
          kv_sems, xl_s, xl_r, xr_s, xr_r, a_s, a_r, b_s, b_r):
    my = lax.axis_index("i")
    right = lax.rem(my + 1, N_DEV)
    left = lax.rem(my + N_DEV - 1, N_DEV)
    hstart = H_PER * my

    kv_copies = []
    for hh in range(H_PER):
        c = pltpu.make_async_copy(
            k_hbm.at[0, pl.ds(0, K_ROWS), hstart + hh, :],
            kf32.at[hh], kv_sems.at[hh])
        c.start()
        kv_copies.append(c)
        c = pltpu.make_async_copy(
            v_hbm.at[0, pl.ds(0, K_ROWS), hstart + hh, :],
            vf32.at[hh], kv_sems.at[H_PER + hh])
        c.start()
        kv_copies.append(c)

    xb[...] = x_ref[...].astype(BF)

    barrier = pltpu.get_barrier_semaphore()
    for nbr in (left, right):
        pl.semaphore_signal(barrier, inc=1, device_id=(nbr,),
                            device_id_type=pl.DeviceIdType.MESH)
    pl.semaphore_wait(barrier, 2)

    send_waits = []

    def rsend(src, dst, ssem, rsem, dev):
        dsc = pltpu.make_async_remote_copy(
            src_ref=src, dst_ref=dst, send_sem=ssem, recv_sem=rsem,
            device_id=(dev,), device_id_type=pl.DeviceIdType.MESH)
        dsc.start()
        send_waits.append(dsc)

    def rwait(dst, ssem, rsem, src_dev):
        pltpu.make_async_remote_copy(
            src_ref=dst, dst_ref=dst, send_sem=ssem, recv_sem=rsem,
            device_id=(src_dev,), device_id_type=pl.DeviceIdType.MESH,
        ).wait_recv()

    def blk(d):
        return lax.rem(my + d, N_DEV)

    def hs(hi):
        return pl.ds(hi * HALF, HALF)

    def sidx(t, hi):
        return 2 * t + hi

    for hi in range(2):
        rsend(xb.at[hs(hi)], xl.at[1, hs(hi)],
              xl_s.at[sidx(1, hi)], xl_r.at[sidx(1, hi)], right)
        rsend(xb.at[hs(hi)], xr.at[1, hs(hi)],
              xr_s.at[sidx(1, hi)], xr_r.at[sidx(1, hi)], left)
    for t in (1, 2):
        for hi in range(2):
            rwait(xl.at[t, hs(hi)], xl_s.at[sidx(t, hi)],
                  xl_r.at[sidx(t, hi)], left)
            rsend(xl.at[t, hs(hi)], xl.at[t + 1, hs(hi)],
                  xl_s.at[sidx(t + 1, hi)], xl_r.at[sidx(t + 1, hi)], right)
            rwait(xr.at[t, hs(hi)], xr_s.at[sidx(t, hi)],
                  xr_r.at[sidx(t, hi)], right)
            rsend(xr.at[t, hs(hi)], xr.at[t + 1, hs(hi)],
                  xr_s.at[sidx(t + 1, hi)], xr_r.at[sidx(t + 1, hi)], left)
        if t == 1:
            wqb[...] = wq_ref[...].astype(BF)
            wob[...] = wo_ref[...].astype(BF)
    for hi in range(2):
        rwait(xl.at[3, hs(hi)], xl_s.at[sidx(3, hi)],
              xl_r.at[sidx(3, hi)], left)
        rwait(xr.at[3, hs(hi)], xr_s.at[sidx(3, hi)],
              xr_r.at[sidx(3, hi)], right)
        rsend(xr.at[3, hs(hi)], xr.at[4, hs(hi)],
              xr_s.at[sidx(4, hi)], xr_r.at[sidx(4, hi)], left)

    for c in kv_copies:
        c.wait()
    kb[...] = kf32[...].astype(BF)
    vb[...] = vf32[...].astype(BF)

    rows = lax.broadcasted_iota(jnp.int32, (SQ_BLK, WIN), 0)
    cols = lax.broadcasted_iota(jnp.int32, (SQ_BLK, WIN), 1)
    d = rows - cols
    mm_add = jnp.where((d <= 0) & (d >= -256), 0.0, NEG).astype(jnp.float32)
    m0_add = jnp.where(jnp.abs(d) <= 128, 0.0, NEG).astype(jnp.float32)
    BAND = 384
    mm_t = mm_add[0:HALF, 0:BAND]
    m0_t = m0_add[0:HALF, 0:BAND]
    mm_b = mm_add[HALF:SQ_BLK, HALF:WIN]
    m0_b = m0_add[HALF:SQ_BLK, 0:BAND]

    def contribution(x_j, j):
        q = jnp.dot(x_j, wqb[...], preferred_element_type=jnp.float32)
        qb = (q * SCALE).astype(BF)
        start = pl.multiple_of(jnp.maximum(256 * j - 128, 0), 128)
        boff = jnp.where(j == 0, 0, HALF)
        start_b = pl.multiple_of(start + boff, 128)
        madd_t = jnp.where(j == 0, m0_t, mm_t)
        madd_b = jnp.where(j == 0, m0_b, mm_b)
        ctxs = []
        for hh in range(H_PER):
            ctx_halves = []
            for (r0, mad, st) in ((0, madd_t, start),
                                  (HALF, madd_b, start_b)):
                qh = qb[r0:r0 + HALF, hh * DH:(hh + 1) * DH]
                kh = kb[hh, pl.ds(st, BAND), :]
                s = lax.dot_general(qh, kh, (((1,), (1,)), ((), ())),
                                    preferred_element_type=jnp.float32)
                p = jnp.exp(s + mad)
                vh = vb[hh, pl.ds(st, BAND), :]
                num = jnp.dot(p.astype(BF), vh,
                              preferred_element_type=jnp.float32)
                ctx_halves.append(num / jnp.sum(p, axis=-1, keepdims=True))
            ctxs.append(jnp.concatenate(ctx_halves, axis=0))
        ctx = jnp.concatenate(ctxs, axis=1).astype(BF)
        return jnp.dot(ctx, wob[...], preferred_element_type=jnp.float32)

    def chain_stage(acc, s, ssems, rsems, from_dev, to_dev, c):
        for hi in range(2):
            rwait(acc.at[s, hs(hi)], ssems.at[sidx(s, hi)],
                  rsems.at[sidx(s, hi)], from_dev)
            lo, hi_ = hi * HALF, (hi + 1) * HALF
            acc[s, lo:hi_, :] = (acc[s, lo:hi_, :].astype(jnp.float32)
                                 + c[lo:hi_, :]).astype(BF)
            rsend(acc.at[s, hs(hi)], acc.at[s + 1, hs(hi)],
                  ssems.at[sidx(s + 1, hi)], rsems.at[sidx(s + 1, hi)],
                  to_dev)

    bacc[0, :, :] = contribution(xr[3], blk(3)).astype(BF)
    for hi in range(2):
        rsend(bacc.at[0, hs(hi)], bacc.at[1, hs(hi)],
              b_s.at[sidx(1, hi)], b_r.at[sidx(1, hi)], right)

    for hi in range(2):
        rwait(xr.at[4, hs(hi)], xr_s.at[sidx(4, hi)],
              xr_r.at[sidx(4, hi)], right)
    aacc[0, :, :] = contribution(xr[4], blk(4)).astype(BF)
    for hi in range(2):
        rsend(aacc.at[0, hs(hi)], aacc.at[1, hs(hi)],
              a_s.at[sidx(1, hi)], a_r.at[sidx(1, hi)], left)


    c2 = contribution(xr[2], blk(2))
    chain_stage(bacc, 1, b_s, b_r, left, right, c2)

    c5 = contribution(xl[3], blk(5))
    chain_stage(aacc, 1, a_s, a_r, right, left, c5)

    c1 = contribution(xr[1], blk(1))
    chain_stage(bacc, 2, b_s, b_r, left, right, c1)

    c6 = contribution(xl[2], blk(6))
    chain_stage(aacc, 2, a_s, a_r, right, left, c6)

    c7 = contribution(xl[1], blk(7))
    chain_stage(aacc, 3, a_s, a_r, right, left, c7)

    c_own = contribution(xb[...], my)
    for hi in range(2):
        rwait(bacc.at[3, hs(hi)], b_s.at[sidx(3, hi)],
              b_r.at[sidx(3, hi)], left)
        rwait(aacc.at[4, hs(hi)], a_s.at[sidx(4, hi)],
              a_r.at[sidx(4, hi)], right)
    out_ref[...] = (aacc[4, :, :].astype(jnp.float32)
                    + bacc[3, :, :].astype(jnp.float32) + c_own)

    for dsc in send_waits:
        dsc.wait_send()


def kernel(x, Wq, K_ext, V_ext, Wo):
    slot = (SQ_BLK, D_MODEL)
    out = pl.pallas_call(
        _body,
        out_shape=jax.ShapeDtypeStruct(slot, jnp.float32),
        in_specs=[
            pl.BlockSpec(memory_space=pltpu.VMEM),
            pl.BlockSpec(memory_space=pltpu.VMEM),
            pl.BlockSpec(memory_space=pl.ANY),
            pl.BlockSpec(memory_space=pl.ANY),
            pl.BlockSpec(memory_space=pltpu.VMEM),
        ],
        out_specs=pl.BlockSpec(memory_space=pltpu.VMEM),
        scratch_shapes=[
            pltpu.VMEM((H_PER, K_ROWS, DH), jnp.float32),
            pltpu.VMEM((H_PER, K_ROWS, DH), jnp.float32),
            pltpu.VMEM((H_PER, K_ROWS, DH), BF),
            pltpu.VMEM((H_PER, K_ROWS, DH), BF),
            pltpu.VMEM(slot, BF),
            pltpu.VMEM((D_MODEL, D_MODEL), BF),
            pltpu.VMEM((D_MODEL, D_MODEL), BF),
            pltpu.VMEM((4,) + slot, BF),
            pltpu.VMEM((5,) + slot, BF),
            pltpu.VMEM((5,) + slot, BF),
            pltpu.VMEM((4,) + slot, BF),
            pltpu.SemaphoreType.DMA((2 * H_PER,)),
            pltpu.SemaphoreType.DMA((8,)),
            pltpu.SemaphoreType.DMA((8,)),
            pltpu.SemaphoreType.DMA((10,)),
            pltpu.SemaphoreType.DMA((10,)),
            pltpu.SemaphoreType.DMA((10,)),
            pltpu.SemaphoreType.DMA((10,)),
            pltpu.SemaphoreType.DMA((8,)),
            pltpu.SemaphoreType.DMA((8,)),
        ],
        compiler_params=pltpu.CompilerParams(
            collective_id=0, vmem_limit_bytes=100 * 1024 * 1024),
    )(x[0], Wq, K_ext, V_ext, Wo)
    return out[None]


# device time: 66467 ns/iter; 1.0816x vs baseline; 1.0816x over previous
import jax
import jax.numpy as jnp
from jax import lax
from jax.experimental import pallas as pl
from jax.experimental.pallas import tpu as pltpu

N_DEV = 8
SQ_BLK = 256
HALF = SQ_BLK // 2
D_MODEL = 1024
H_PER = 8
DH = 128
WIN = 512
K_ROWS = 2176
SCALE = 0.08838834764831843
NEG = -1e9
BF = jnp.bfloat16


def _body(x_ref, wq_ref, k_hbm, v_hbm, wo_ref, out_ref,
          kf32, vf32, kb, vb, xb, wqb, wob, xl, xr, aacc, bacc,
          kv_sems, xl_s, xl_r, xr_s, xr_r, a_s, a_r, b_s, b_r):
    my = lax.axis_index("i")
    right = lax.rem(my + 1, N_DEV)
    left = lax.rem(my + N_DEV - 1, N_DEV)
    hstart = H_PER * my

    kv_copies = []
    for hh in range(H_PER):
        c = pltpu.make_async_copy(
            k_hbm.at[0, pl.ds(0, K_ROWS), hstart + hh, :],
            kf32.at[hh], kv_sems.at[hh])
        c.start()
        kv_copies.append(c)
        c = pltpu.make_async_copy(
            v_hbm.at[0, pl.ds(0, K_ROWS), hstart + hh, :],
            vf32.at[hh], kv_sems.at[H_PER + hh])
        c.start()
        kv_copies.append(c)

    xb[...] = x_ref[...].astype(BF)

    barrier = pltpu.get_barrier_semaphore()
    for nbr in (left, right):
        pl.semaphore_signal(barrier, inc=1, device_id=(nbr,),
                            device_id_type=pl.DeviceIdType.MESH)
    pl.semaphore_wait(barrier, 2)

    send_waits = []

    def rsend(src, dst, ssem, rsem, dev):
        dsc = pltpu.make_async_remote_copy(
            src_ref=src, dst_ref=dst, send_sem=ssem, recv_sem=rsem,
            device_id=(dev,), device_id_type=pl.DeviceIdType.MESH)
        dsc.start()
        send_waits.append(dsc)

    def rwait(dst, ssem, rsem, src_dev):
        pltpu.make_async_remote_copy(
            src_ref=dst, dst_ref=dst, send_sem=ssem, recv_sem=rsem,
            device_id=(src_dev,), device_id_type=pl.DeviceIdType.MESH,
        ).wait_recv()

    def blk(d):
        return lax.rem(my + d, N_DEV)

    def hs(hi):
        return pl.ds(hi * HALF, HALF)

    def sidx(t, hi):
        return 2 * t + hi

    for hi in range(2):
        rsend(xb.at[hs(hi)], xl.at[1, hs(hi)],
              xl_s.at[sidx(1, hi)], xl_r.at[sidx(1, hi)], right)
        rsend(xb.at[hs(hi)], xr.at[1, hs(hi)],
              xr_s.at[sidx(1, hi)], xr_r.at[sidx(1, hi)], left)
    for t in (1, 2):
        for hi in range(2):
            rwait(xl.at[t, hs(hi)], xl_s.at[sidx(t, hi)],
                  xl_r.at[sidx(t, hi)], left)
            rsend(xl.at[t, hs(hi)], xl.at[t + 1, hs(hi)],
                  xl_s.at[sidx(t + 1, hi)], xl_r.at[sidx(t + 1, hi)], right)
            rwait(xr.at[t, hs(hi)], xr_s.at[sidx(t, hi)],
                  xr_r.at[sidx(t, hi)], right)
            rsend(xr.at[t, hs(hi)], xr.at[t + 1, hs(hi)],
                  xr_s.at[sidx(t + 1, hi)], xr_r.at[sidx(t + 1, hi)], left)
        if t == 1:
            wqb[...] = wq_ref[...].astype(BF)
            wob[...] = wo_ref[...].astype(BF)
    for hi in range(2):
        rwait(xl.at[3, hs(hi)], xl_s.at[sidx(3, hi)],
              xl_r.at[sidx(3, hi)], left)
        rwait(xr.at[3, hs(hi)], xr_s.at[sidx(3, hi)],
              xr_r.at[sidx(3, hi)], right)
        rsend(xr.at[3, hs(hi)], xr.at[4, hs(hi)],
              xr_s.at[sidx(4, hi)], xr_r.at[sidx(4, hi)], left)

    for c in kv_copies:
        c.wait()
    kb[...] = kf32[...].astype(BF)
    vb[...] = vf32[...].astype(BF)

    rows = lax.broadcasted_iota(jnp.int32, (SQ_BLK, WIN), 0)
    cols = lax.broadcasted_iota(jnp.int32, (SQ_BLK, WIN), 1)
    d = rows - cols
    mm_add = jnp.where((d <= 0) & (d >= -256), 0.0, NEG).astype(jnp.float32)
    m0_add = jnp.where(jnp.abs(d) <= 128, 0.0, NEG).astype(jnp.float32)

    def contribution(x_j, j):
        q = jnp.dot(x_j, wqb[...], preferred_element_type=jnp.float32)
        qb = (q * SCALE).astype(BF)
        start = pl.multiple_of(jnp.maximum(256 * j - 128, 0), 128)
        madd = jnp.where(j == 0, m0_add, mm_add)
        ctxs = []
        for hh in range(H_PER):
            qh = qb[:, hh * DH:(hh + 1) * DH]
            kh = kb[hh, pl.ds(start, WIN), :]
            s = lax.dot_general(qh, kh, (((1,), (1,)), ((), ())),
                                preferred_element_type=jnp.float32)
            p = jnp.exp(s + madd)
            vh = vb[hh, pl.ds(start, WIN), :]
            num = jnp.dot(p.astype(BF), vh,
                          preferred_element_type=jnp.float32)
            ctxs.append(num / jnp.sum(p, axis=-1, keepdims=True))
        ctx = jnp.concatenate(ctxs, axis=1).astype(BF)
        return jnp.dot(ctx, wob[...], preferred_element_type=jnp.float32)

    def chain_stage(acc, s, ssems, rsems, from_dev, to_dev, c):
        for hi in range(2):
            rwait(acc.at[s, hs(hi)], ssems.at[sidx(s, hi)],
                  rsems.at[sidx(s, hi)], from_dev)
            lo, hi_ = hi * HALF, (hi + 1) * HALF
            acc[s, lo:hi_, :] = (acc[s, lo:hi_, :].astype(jnp.float32)
                                 + c[lo:hi_, :]).astype(BF)
            rsend(acc.at[s, hs(hi)], acc.at[s + 1, hs(hi)],
                  ssems.at[sidx(s + 1, hi)], rsems.at[sidx(s + 1, hi)],
                  to_dev)

    bacc[0, :, :] = contribution(xr[3], blk(3)).astype(BF)
    for hi in range(2):
        rsend(bacc.at[0, hs(hi)], bacc.at[1, hs(hi)],
              b_s.at[sidx(1, hi)], b_r.at[sidx(1, hi)], right)

    for hi in range(2):
        rwait(xr.at[4, hs(hi)], xr_s.at[sidx(4, hi)],
              xr_r.at[sidx(4, hi)], right)
    aacc[0, :, :] = contribution(xr[4], blk(4)).astype(BF)
    for hi in range(2):
        rsend(aacc.at[0, hs(hi)], aacc.at[1, hs(hi)],
              a_s.at[sidx(1, hi)], a_r.at[sidx(1, hi)], left)


    c2 = contribution(xr[2], blk(2))
    chain_stage(bacc, 1, b_s, b_r, left, right, c2)

    c5 = contribution(xl[3], blk(5))
    chain_stage(aacc, 1, a_s, a_r, right, left, c5)

    c1 = contribution(xr[1], blk(1))
    chain_stage(bacc, 2, b_s, b_r, left, right, c1)

    c6 = contribution(xl[2], blk(6))
    chain_stage(aacc, 2, a_s, a_r, right, left, c6)

    c7 = contribution(xl[1], blk(7))
    chain_stage(aacc, 3, a_s, a_r, right, left, c7)

    c_own = contribution(xb[...], my)
    for hi in range(2):
        rwait(bacc.at[3, hs(hi)], b_s.at[sidx(3, hi)],
              b_r.at[sidx(3, hi)], left)
        rwait(aacc.at[4, hs(hi)], a_s.at[sidx(4, hi)],
              a_r.at[sidx(4, hi)], right)
    out_ref[...] = (aacc[4, :, :].astype(jnp.float32)
                    + bacc[3, :, :].astype(jnp.float32) + c_own)

    for dsc in send_waits:
        dsc.wait_send()


def kernel(x, Wq, K_ext, V_ext, Wo):
    slot = (SQ_BLK, D_MODEL)
    out = pl.pallas_call(
        _body,
        out_shape=jax.ShapeDtypeStruct(slot, jnp.float32),
        in_specs=[
            pl.BlockSpec(memory_space=pltpu.VMEM),
            pl.BlockSpec(memory_space=pltpu.VMEM),
            pl.BlockSpec(memory_space=pl.ANY),
            pl.BlockSpec(memory_space=pl.ANY),
            pl.BlockSpec(memory_space=pltpu.VMEM),
        ],
        out_specs=pl.BlockSpec(memory_space=pltpu.VMEM),
        scratch_shapes=[
            pltpu.VMEM((H_PER, K_ROWS, DH), jnp.float32),
            pltpu.VMEM((H_PER, K_ROWS, DH), jnp.float32),
            pltpu.VMEM((H_PER, K_ROWS, DH), BF),
            pltpu.VMEM((H_PER, K_ROWS, DH), BF),
            pltpu.VMEM(slot, BF),
            pltpu.VMEM((D_MODEL, D_MODEL), BF),
            pltpu.VMEM((D_MODEL, D_MODEL), BF),
            pltpu.VMEM((4,) + slot, BF),
            pltpu.VMEM((5,) + slot, BF),
            pltpu.VMEM((5,) + slot, BF),
            pltpu.VMEM((4,) + slot, BF),
            pltpu.SemaphoreType.DMA((2 * H_PER,)),
            pltpu.SemaphoreType.DMA((8,)),
            pltpu.SemaphoreType.DMA((8,)),
            pltpu.SemaphoreType.DMA((10,)),
            pltpu.SemaphoreType.DMA((10,)),
            pltpu.SemaphoreType.DMA((10,)),
            pltpu.SemaphoreType.DMA((10,)),
            pltpu.SemaphoreType.DMA((8,)),
            pltpu.SemaphoreType.DMA((8,)),
        ],
        compiler_params=pltpu.CompilerParams(
            collective_id=0, vmem_limit_bytes=100 * 1024 * 1024),
    )(x[0], Wq, K_ext, V_ext, Wo)
    return out[None]
